# baseline (speedup 1.0000x reference)
import functools

import jax
import jax.numpy as jnp
from jax import lax
from jax.experimental import pallas as pl
from jax.experimental.pallas import tpu as pltpu
from jax.experimental.pallas import tpu_sc as plsc

NC, NS = 2, 16
NW = NC * NS
N = 10000
D = 128
H = 256
E = 320000
CH = 96
K_CH = 106
E_W = K_CH * CH
ACC_ROWS = 10112
STRIPE = 632
M_BLK = 1000
GRID_M = N // M_BLK
TB = 1024

@functools.cache
def _sc_mesh():
    return plsc.VectorSubcoreMesh(
        core_axis_name="c", subcore_axis_name="s",
        num_cores=NC, num_subcores=NS)


def _segsum_body(table, srcx, dstx, out, src_v, dst_v, rows_v, acc,
                 sem0, sem1):
    cid = lax.axis_index("c")
    sid = lax.axis_index("s")
    wid = cid * NS + sid

    zeros16 = jnp.zeros((16,), jnp.float32)

    def zrow(r, carry):
        for c in range(8):
            rows_v[0, r, pl.ds(c * 16, 16)] = zeros16
        return carry

    lax.fori_loop(0, CH, zrow, 0)
    zb = sid * STRIPE
    nfull = STRIPE // CH
    for j in range(nfull):
        pltpu.sync_copy(rows_v.at[0], acc.at[pl.ds(zb + j * CH, CH)])
    pltpu.sync_copy(rows_v.at[0].at[pl.ds(0, STRIPE - nfull * CH)],
                    acc.at[pl.ds(zb + nfull * CH, STRIPE - nfull * CH)])

    pltpu.sync_copy(srcx.at[wid], src_v)
    pltpu.sync_copy(dstx.at[wid], dst_v)
    plsc.subcore_barrier()

    pltpu.async_copy(table.at[src_v.at[pl.ds(0, CH)]], rows_v.at[0], sem0)
    pltpu.async_copy(table.at[src_v.at[pl.ds(CH, CH)]], rows_v.at[1], sem1)

    def body(i, carry):
        k = 2 * i
        pltpu.make_async_copy(table.at[src_v.at[pl.ds(k * CH, CH)]],
                              rows_v.at[0], sem0).wait()
        pltpu.sync_copy(rows_v.at[0], acc.at[dst_v.at[k]], add=True)

        @pl.when(k + 2 < K_CH)
        def _():
            pltpu.async_copy(table.at[src_v.at[pl.ds((k + 2) * CH, CH)]],
                             rows_v.at[0], sem0)

        pltpu.make_async_copy(table.at[src_v.at[pl.ds((k + 1) * CH, CH)]],
                              rows_v.at[1], sem1).wait()
        pltpu.sync_copy(rows_v.at[1], acc.at[dst_v.at[k + 1]], add=True)

        @pl.when(k + 3 < K_CH)
        def _():
            pltpu.async_copy(table.at[src_v.at[pl.ds((k + 3) * CH, CH)]],
                             rows_v.at[1], sem1)

        return carry

    lax.fori_loop(0, K_CH // 2, body, 0)
    plsc.subcore_barrier()

    ob = sid * STRIPE
    pltpu.sync_copy(acc.at[pl.ds(ob, STRIPE)],
                    out.at[cid, pl.ds(ob, STRIPE)])


@functools.cache
def _segsum():
    return pl.kernel(
        _segsum_body,
        out_type=jax.ShapeDtypeStruct((NC, ACC_ROWS, D), jnp.float32),
        mesh=_sc_mesh(),
        scratch_types=[
            pltpu.VMEM((E_W,), jnp.int32),
            pltpu.VMEM((K_CH, CH), jnp.int32),
            pltpu.VMEM((2, CH, D), jnp.float32),
            pltpu.VMEM_SHARED((ACC_ROWS, D), jnp.float32),
            pltpu.SemaphoreType.DMA,
            pltpu.SemaphoreType.DMA,
        ],
    )


def _target_gather_body(table, idx, out, idx_v, rows_v, sem):
    cid = lax.axis_index("c")
    sid = lax.axis_index("s")
    wid = cid * NS + sid
    base = wid * (TB // NW)
    pltpu.sync_copy(idx.at[pl.ds(base, TB // NW)], idx_v)
    pltpu.async_copy(table.at[idx_v], rows_v, sem).wait()
    pltpu.sync_copy(rows_v, out.at[pl.ds(base, TB // NW)])


@functools.cache
def _target_gather():
    return pl.kernel(
        _target_gather_body,
        out_type=jax.ShapeDtypeStruct((TB, D), jnp.float32),
        mesh=_sc_mesh(),
        scratch_types=[
            pltpu.VMEM((TB // NW,), jnp.int32),
            pltpu.VMEM((TB // NW, D), jnp.float32),
            pltpu.SemaphoreType.DMA,
        ],
    )


def _l1_body(v_ref, p_ref, w1a_ref, w1b_ref, x1_ref, agg_ref):
    agg = p_ref[0] + p_ref[1]
    agg_ref[...] = agg
    x1_ref[...] = (
        jnp.dot(v_ref[...], w1a_ref[...], preferred_element_type=jnp.float32)
        + jnp.dot(agg, w1b_ref[...], preferred_element_type=jnp.float32))


def _l2_body(x1_ref, agg0_ref, q_ref, w1a_ref, w1b_ref, w2a_ref, w2b_ref,
             b2_ref, x2lo_ref, x2hi_ref):
    agg0b = q_ref[0] + q_ref[1]
    agg1 = (
        jnp.dot(agg0_ref[...], w1a_ref[...], preferred_element_type=jnp.float32)
        + jnp.dot(agg0b, w1b_ref[...], preferred_element_type=jnp.float32))
    x1 = x1_ref[...]
    h = (jnp.dot(x1, w2a_ref[...], preferred_element_type=jnp.float32)
         + jnp.dot(agg1, w2b_ref[...], preferred_element_type=jnp.float32)
         + b2_ref[...])
    h = jnp.maximum(h, 0.0)
    h = h / (jnp.sqrt(jnp.sum(h * h, axis=1, keepdims=True)) + 1e-6)
    x2 = h + x1
    x2lo_ref[...] = x2[:, :D]
    x2hi_ref[...] = x2[:, D:]


def _l3_body(x2lo_ref, x2hi_ref, rlo_ref, rhi_ref, w3alo_ref, w3ahi_ref,
             w3blo_ref, w3bhi_ref, b3_ref, w4p_ref, y_ref):
    agg_lo = rlo_ref[0] + rlo_ref[1]
    agg_hi = rhi_ref[0] + rhi_ref[1]
    h = (jnp.dot(x2lo_ref[...], w3alo_ref[...], preferred_element_type=jnp.float32)
         + jnp.dot(x2hi_ref[...], w3ahi_ref[...], preferred_element_type=jnp.float32)
         + jnp.dot(agg_lo, w3blo_ref[...], preferred_element_type=jnp.float32)
         + jnp.dot(agg_hi, w3bhi_ref[...], preferred_element_type=jnp.float32)
         + b3_ref[...])
    h = jnp.maximum(h, 0.0)
    h = h / (jnp.sqrt(jnp.sum(h * h, axis=1, keepdims=True)) + 1e-6)
    y_ref[...] = jnp.dot(h, w4p_ref[...], preferred_element_type=jnp.float32)


def _row_spec(d):
    return pl.BlockSpec((M_BLK, d), lambda m: (m, 0))


def _pair_spec(d):
    return pl.BlockSpec((NC, M_BLK, d), lambda m: (0, m, 0))


def _full_spec(r, c):
    return pl.BlockSpec((r, c), lambda m: (0, 0))


_l1 = pl.pallas_call(
    _l1_body,
    grid=(GRID_M,),
    in_specs=[_row_spec(D), _pair_spec(D), _full_spec(D, H), _full_spec(D, H)],
    out_specs=[_row_spec(H), _row_spec(D)],
    out_shape=[jax.ShapeDtypeStruct((N, H), jnp.float32),
               jax.ShapeDtypeStruct((N, D), jnp.float32)],
)

_l2 = pl.pallas_call(
    _l2_body,
    grid=(GRID_M,),
    in_specs=[_row_spec(H), _row_spec(D), _pair_spec(D),
              _full_spec(D, H), _full_spec(D, H),
              _full_spec(H, H), _full_spec(H, H), _full_spec(1, H)],
    out_specs=[_row_spec(D), _row_spec(D)],
    out_shape=[jax.ShapeDtypeStruct((N, D), jnp.float32),
               jax.ShapeDtypeStruct((N, D), jnp.float32)],
)

_l3 = pl.pallas_call(
    _l3_body,
    grid=(GRID_M,),
    in_specs=[_row_spec(D), _row_spec(D), _pair_spec(D), _pair_spec(D),
              _full_spec(D, H), _full_spec(D, H), _full_spec(D, H),
              _full_spec(D, H), _full_spec(1, H), _full_spec(H, D)],
    out_specs=_row_spec(D),
    out_shape=jax.ShapeDtypeStruct((N, D), jnp.float32),
)


def kernel(edges, vertices, target_idx, W1, W2, b2, W3, b3, W4):
    src = edges[:, 1].astype(jnp.int32)
    dst = edges[:, 0].astype(jnp.int32)
    pad = NW * E_W - E
    srcx = (jnp.concatenate([src, jnp.zeros((pad,), jnp.int32)])
            .reshape(K_CH, NW, CH).transpose(1, 0, 2).reshape(NW, E_W))
    chunk_off = (jnp.arange(E_W, dtype=jnp.int32) % 16) * N
    srcx = srcx + chunk_off[None, :]
    dstx = (jnp.concatenate([dst, jnp.full((pad,), N, jnp.int32)])
            .reshape(K_CH, NW, CH).transpose(1, 0, 2))

    w1t = W1.T
    w1a, w1b = w1t[:D], w1t[D:]
    w2t = W2.T
    w2a, w2b = w2t[:H], w2t[H:]
    w3t = W3.T
    w3a_lo, w3a_hi = w3t[:D], w3t[D:H]
    w3b_lo, w3b_hi = w3t[H:H + D], w3t[H + D:]
    w4p = jnp.zeros((D, H), jnp.float32).at[:2].set(W4).T

    _seg = _segsum()

    def segsum(t, s, d):
        return _seg(jnp.concatenate([t] * 16), s, d)

    p = segsum(vertices, srcx, dstx)
    x1, agg0 = _l1(vertices, p, w1a, w1b)
    q = segsum(agg0, srcx, dstx)
    x2lo, x2hi = _l2(x1, agg0, q, w1a, w1b, w2a, w2b, b2.reshape(1, H))
    rlo = segsum(x2lo, srcx, dstx)
    rhi = segsum(x2hi, srcx, dstx)
    y = _l3(x2lo, x2hi, rlo, rhi, w3a_lo, w3a_hi, w3b_lo, w3b_hi,
            b3.reshape(1, H), w4p)
    yt = _target_gather()(y, target_idx.astype(jnp.int32))
    return yt[:, :2][None, :, :]


# speedup vs baseline: 1.0218x; 1.0218x over previous
import functools

import jax
import jax.numpy as jnp
from jax import lax
from jax.experimental import pallas as pl
from jax.experimental.pallas import tpu as pltpu
from jax.experimental.pallas import tpu_sc as plsc

NC, NS = 2, 16
NW = NC * NS
N = 10000
D = 128
H = 256
E = 320000
CH = 96
K_CH = 106
E_W = K_CH * CH
ACC_ROWS = 10112
STRIPE = 632
M_BLK = 1000
GRID_M = N // M_BLK
TB = 1024

@functools.cache
def _sc_mesh():
    return plsc.VectorSubcoreMesh(
        core_axis_name="c", subcore_axis_name="s",
        num_cores=NC, num_subcores=NS)


def _segsum_body(table, srcx, dstx, out, src_v, dst_v, rows_v, acc,
                 sem0, sem1):
    cid = lax.axis_index("c")
    sid = lax.axis_index("s")
    wid = cid * NS + sid

    zeros16 = jnp.zeros((16,), jnp.float32)

    def zrow(r, carry):
        for c in range(8):
            rows_v[0, r, pl.ds(c * 16, 16)] = zeros16
        return carry

    lax.fori_loop(0, CH, zrow, 0)
    zb = sid * STRIPE
    nfull = STRIPE // CH
    for j in range(nfull):
        pltpu.sync_copy(rows_v.at[0], acc.at[pl.ds(zb + j * CH, CH)])
    pltpu.sync_copy(rows_v.at[0].at[pl.ds(0, STRIPE - nfull * CH)],
                    acc.at[pl.ds(zb + nfull * CH, STRIPE - nfull * CH)])

    pltpu.sync_copy(srcx.at[wid], src_v)
    pltpu.sync_copy(dstx.at[wid], dst_v)
    plsc.subcore_barrier()

    pltpu.async_copy(table.at[src_v.at[pl.ds(0, CH)]], rows_v.at[0], sem0)
    pltpu.async_copy(table.at[src_v.at[pl.ds(CH, CH)]], rows_v.at[1], sem1)

    def body(i, carry):
        k = 2 * i
        pltpu.make_async_copy(table.at[src_v.at[pl.ds(k * CH, CH)]],
                              rows_v.at[0], sem0).wait()
        pltpu.sync_copy(rows_v.at[0], acc.at[dst_v.at[k]], add=True)

        @pl.when(k + 2 < K_CH)
        def _():
            pltpu.async_copy(table.at[src_v.at[pl.ds((k + 2) * CH, CH)]],
                             rows_v.at[0], sem0)

        pltpu.make_async_copy(table.at[src_v.at[pl.ds((k + 1) * CH, CH)]],
                              rows_v.at[1], sem1).wait()
        pltpu.sync_copy(rows_v.at[1], acc.at[dst_v.at[k + 1]], add=True)

        @pl.when(k + 3 < K_CH)
        def _():
            pltpu.async_copy(table.at[src_v.at[pl.ds((k + 3) * CH, CH)]],
                             rows_v.at[1], sem1)

        return carry

    lax.fori_loop(0, K_CH // 2, body, 0)
    plsc.subcore_barrier()

    ob = sid * STRIPE
    pltpu.sync_copy(acc.at[pl.ds(ob, STRIPE)],
                    out.at[cid, pl.ds(ob, STRIPE)])


@functools.cache
def _segsum():
    return pl.kernel(
        _segsum_body,
        out_type=jax.ShapeDtypeStruct((NC, ACC_ROWS, D), jnp.float32),
        mesh=_sc_mesh(),
        scratch_types=[
            pltpu.VMEM((E_W,), jnp.int32),
            pltpu.VMEM((K_CH, CH), jnp.int32),
            pltpu.VMEM((2, CH, D), jnp.float32),
            pltpu.VMEM_SHARED((ACC_ROWS, D), jnp.float32),
            pltpu.SemaphoreType.DMA,
            pltpu.SemaphoreType.DMA,
        ],
    )


def _target_gather_body(table, idx, out, idx_v, rows_v, sem):
    cid = lax.axis_index("c")
    sid = lax.axis_index("s")
    wid = cid * NS + sid
    base = wid * (TB // NW)
    pltpu.sync_copy(idx.at[pl.ds(base, TB // NW)], idx_v)
    pltpu.async_copy(table.at[idx_v], rows_v, sem).wait()
    pltpu.sync_copy(rows_v, out.at[pl.ds(base, TB // NW)])


@functools.cache
def _target_gather():
    return pl.kernel(
        _target_gather_body,
        out_type=jax.ShapeDtypeStruct((TB, D), jnp.float32),
        mesh=_sc_mesh(),
        scratch_types=[
            pltpu.VMEM((TB // NW,), jnp.int32),
            pltpu.VMEM((TB // NW, D), jnp.float32),
            pltpu.SemaphoreType.DMA,
        ],
    )


def _l1_body(v_ref, p_ref, w1a_ref, w1b_ref, x1_ref, agg_ref):
    agg = p_ref[0] + p_ref[1]
    agg_ref[...] = agg
    x1_ref[...] = (
        jnp.dot(v_ref[...], w1a_ref[...], preferred_element_type=jnp.float32)
        + jnp.dot(agg, w1b_ref[...], preferred_element_type=jnp.float32))


def _l2_body(x1_ref, agg0_ref, q_ref, w1a_ref, w1b_ref, w2a_ref, w2b_ref,
             b2_ref, x2lo_ref, x2hi_ref):
    agg0b = q_ref[0] + q_ref[1]
    agg1 = (
        jnp.dot(agg0_ref[...], w1a_ref[...], preferred_element_type=jnp.float32)
        + jnp.dot(agg0b, w1b_ref[...], preferred_element_type=jnp.float32))
    x1 = x1_ref[...]
    h = (jnp.dot(x1, w2a_ref[...], preferred_element_type=jnp.float32)
         + jnp.dot(agg1, w2b_ref[...], preferred_element_type=jnp.float32)
         + b2_ref[...])
    h = jnp.maximum(h, 0.0)
    h = h / (jnp.sqrt(jnp.sum(h * h, axis=1, keepdims=True)) + 1e-6)
    x2 = h + x1
    x2lo_ref[...] = x2[:, :D]
    x2hi_ref[...] = x2[:, D:]


def _l3_body(x2lo_ref, x2hi_ref, rlo_ref, rhi_ref, w3alo_ref, w3ahi_ref,
             w3blo_ref, w3bhi_ref, b3_ref, w4p_ref, y_ref):
    agg_lo = rlo_ref[0] + rlo_ref[1]
    agg_hi = rhi_ref[0] + rhi_ref[1]
    h = (jnp.dot(x2lo_ref[...], w3alo_ref[...], preferred_element_type=jnp.float32)
         + jnp.dot(x2hi_ref[...], w3ahi_ref[...], preferred_element_type=jnp.float32)
         + jnp.dot(agg_lo, w3blo_ref[...], preferred_element_type=jnp.float32)
         + jnp.dot(agg_hi, w3bhi_ref[...], preferred_element_type=jnp.float32)
         + b3_ref[...])
    h = jnp.maximum(h, 0.0)
    h = h / (jnp.sqrt(jnp.sum(h * h, axis=1, keepdims=True)) + 1e-6)
    y_ref[...] = jnp.dot(h, w4p_ref[...], preferred_element_type=jnp.float32)


def _row_spec(d):
    return pl.BlockSpec((M_BLK, d), lambda m: (m, 0))


def _pair_spec(d):
    return pl.BlockSpec((NC, M_BLK, d), lambda m: (0, m, 0))


def _full_spec(r, c):
    return pl.BlockSpec((r, c), lambda m: (0, 0))


_l1 = pl.pallas_call(
    _l1_body,
    grid=(GRID_M,),
    in_specs=[_row_spec(D), _pair_spec(D), _full_spec(D, H), _full_spec(D, H)],
    out_specs=[_row_spec(H), _row_spec(D)],
    out_shape=[jax.ShapeDtypeStruct((N, H), jnp.float32),
               jax.ShapeDtypeStruct((N, D), jnp.float32)],
)

_l2 = pl.pallas_call(
    _l2_body,
    grid=(GRID_M,),
    in_specs=[_row_spec(H), _row_spec(D), _pair_spec(D),
              _full_spec(D, H), _full_spec(D, H),
              _full_spec(H, H), _full_spec(H, H), _full_spec(1, H)],
    out_specs=[_row_spec(D), _row_spec(D)],
    out_shape=[jax.ShapeDtypeStruct((N, D), jnp.float32),
               jax.ShapeDtypeStruct((N, D), jnp.float32)],
)

_l3 = pl.pallas_call(
    _l3_body,
    grid=(GRID_M,),
    in_specs=[_row_spec(D), _row_spec(D), _pair_spec(D), _pair_spec(D),
              _full_spec(D, H), _full_spec(D, H), _full_spec(D, H),
              _full_spec(D, H), _full_spec(1, H), _full_spec(H, D)],
    out_specs=_row_spec(D),
    out_shape=jax.ShapeDtypeStruct((N, D), jnp.float32),
)


def kernel(edges, vertices, target_idx, W1, W2, b2, W3, b3, W4):
    src = edges[:, 1].astype(jnp.int32)
    dst = edges[:, 0].astype(jnp.int32)
    pad = NW * E_W - E
    srcx = (jnp.concatenate([src, jnp.zeros((pad,), jnp.int32)])
            .reshape(K_CH, NW, CH).transpose(1, 0, 2).reshape(NW, E_W))
    chunk_off = (jnp.arange(E_W, dtype=jnp.int32) % 8) * N
    srcx = srcx + chunk_off[None, :]
    dstx = (jnp.concatenate([dst, jnp.full((pad,), N, jnp.int32)])
            .reshape(K_CH, NW, CH).transpose(1, 0, 2))

    w1t = W1.T
    w1a, w1b = w1t[:D], w1t[D:]
    w2t = W2.T
    w2a, w2b = w2t[:H], w2t[H:]
    w3t = W3.T
    w3a_lo, w3a_hi = w3t[:D], w3t[D:H]
    w3b_lo, w3b_hi = w3t[H:H + D], w3t[H + D:]
    w4p = jnp.zeros((D, H), jnp.float32).at[:2].set(W4).T

    _seg = _segsum()

    def segsum(t, s, d):
        return _seg(jnp.concatenate([t, t, t, t, t, t, t, t]), s, d)

    p = segsum(vertices, srcx, dstx)
    x1, agg0 = _l1(vertices, p, w1a, w1b)
    q = segsum(agg0, srcx, dstx)
    x2lo, x2hi = _l2(x1, agg0, q, w1a, w1b, w2a, w2b, b2.reshape(1, H))
    rlo = segsum(x2lo, srcx, dstx)
    rhi = segsum(x2hi, srcx, dstx)
    y = _l3(x2lo, x2hi, rlo, rhi, w3a_lo, w3a_hi, w3b_lo, w3b_hi,
            b3.reshape(1, H), w4p)
    yt = _target_gather()(y, target_idx.astype(jnp.int32))
    return yt[:, :2][None, :, :]


# speedup vs baseline: 1.0842x; 1.0610x over previous
import functools

import jax
import jax.numpy as jnp
from jax import lax
from jax.experimental import pallas as pl
from jax.experimental.pallas import tpu as pltpu
from jax.experimental.pallas import tpu_sc as plsc

NC, NS = 2, 16
NW = NC * NS
N = 10000
D = 128
H = 256
E = 320000
CH = 104
K_CH = 98
E_W = K_CH * CH
ACC_ROWS = 10112
STRIPE = 632
M_BLK = 1000
GRID_M = N // M_BLK
TB = 1024

@functools.cache
def _sc_mesh():
    return plsc.VectorSubcoreMesh(
        core_axis_name="c", subcore_axis_name="s",
        num_cores=NC, num_subcores=NS)


def _segsum_body(table, srcx, dstx, out, src_v, dst_v, rows_v, acc,
                 sem0, sem1):
    cid = lax.axis_index("c")
    sid = lax.axis_index("s")
    wid = cid * NS + sid

    zeros16 = jnp.zeros((16,), jnp.float32)

    def zrow(r, carry):
        for c in range(8):
            rows_v[0, r, pl.ds(c * 16, 16)] = zeros16
        return carry

    lax.fori_loop(0, CH, zrow, 0)
    zb = sid * STRIPE
    nfull = STRIPE // CH
    for j in range(nfull):
        pltpu.sync_copy(rows_v.at[0], acc.at[pl.ds(zb + j * CH, CH)])
    pltpu.sync_copy(rows_v.at[0].at[pl.ds(0, STRIPE - nfull * CH)],
                    acc.at[pl.ds(zb + nfull * CH, STRIPE - nfull * CH)])

    pltpu.sync_copy(srcx.at[wid], src_v)
    pltpu.sync_copy(dstx.at[wid], dst_v)
    plsc.subcore_barrier()

    pltpu.async_copy(table.at[src_v.at[pl.ds(0, CH)]], rows_v.at[0], sem0)
    pltpu.async_copy(table.at[src_v.at[pl.ds(CH, CH)]], rows_v.at[1], sem1)

    def body(i, carry):
        k = 2 * i
        pltpu.make_async_copy(table.at[src_v.at[pl.ds(k * CH, CH)]],
                              rows_v.at[0], sem0).wait()
        pltpu.sync_copy(rows_v.at[0], acc.at[dst_v.at[k]], add=True)

        @pl.when(k + 2 < K_CH)
        def _():
            pltpu.async_copy(table.at[src_v.at[pl.ds((k + 2) * CH, CH)]],
                             rows_v.at[0], sem0)

        pltpu.make_async_copy(table.at[src_v.at[pl.ds((k + 1) * CH, CH)]],
                              rows_v.at[1], sem1).wait()
        pltpu.sync_copy(rows_v.at[1], acc.at[dst_v.at[k + 1]], add=True)

        @pl.when(k + 3 < K_CH)
        def _():
            pltpu.async_copy(table.at[src_v.at[pl.ds((k + 3) * CH, CH)]],
                             rows_v.at[1], sem1)

        return carry

    lax.fori_loop(0, K_CH // 2, body, 0)
    plsc.subcore_barrier()

    ob = sid * STRIPE
    pltpu.sync_copy(acc.at[pl.ds(ob, STRIPE)],
                    out.at[cid, pl.ds(ob, STRIPE)])


@functools.cache
def _segsum():
    return pl.kernel(
        _segsum_body,
        out_type=jax.ShapeDtypeStruct((NC, ACC_ROWS, D), jnp.float32),
        mesh=_sc_mesh(),
        scratch_types=[
            pltpu.VMEM((E_W,), jnp.int32),
            pltpu.VMEM((K_CH, CH), jnp.int32),
            pltpu.VMEM((2, CH, D), jnp.float32),
            pltpu.VMEM_SHARED((ACC_ROWS, D), jnp.float32),
            pltpu.SemaphoreType.DMA,
            pltpu.SemaphoreType.DMA,
        ],
    )


def _target_gather_body(table, idx, out, idx_v, rows_v, sem):
    cid = lax.axis_index("c")
    sid = lax.axis_index("s")
    wid = cid * NS + sid
    base = wid * (TB // NW)
    pltpu.sync_copy(idx.at[pl.ds(base, TB // NW)], idx_v)
    pltpu.async_copy(table.at[idx_v], rows_v, sem).wait()
    pltpu.sync_copy(rows_v, out.at[pl.ds(base, TB // NW)])


@functools.cache
def _target_gather():
    return pl.kernel(
        _target_gather_body,
        out_type=jax.ShapeDtypeStruct((TB, D), jnp.float32),
        mesh=_sc_mesh(),
        scratch_types=[
            pltpu.VMEM((TB // NW,), jnp.int32),
            pltpu.VMEM((TB // NW, D), jnp.float32),
            pltpu.SemaphoreType.DMA,
        ],
    )


def _l1_body(v_ref, p_ref, w1a_ref, w1b_ref, x1_ref, agg_ref):
    agg = p_ref[0] + p_ref[1]
    agg_ref[...] = agg
    x1_ref[...] = (
        jnp.dot(v_ref[...], w1a_ref[...], preferred_element_type=jnp.float32)
        + jnp.dot(agg, w1b_ref[...], preferred_element_type=jnp.float32))


def _l2_body(x1_ref, agg0_ref, q_ref, w1a_ref, w1b_ref, w2a_ref, w2b_ref,
             b2_ref, x2lo_ref, x2hi_ref):
    agg0b = q_ref[0] + q_ref[1]
    agg1 = (
        jnp.dot(agg0_ref[...], w1a_ref[...], preferred_element_type=jnp.float32)
        + jnp.dot(agg0b, w1b_ref[...], preferred_element_type=jnp.float32))
    x1 = x1_ref[...]
    h = (jnp.dot(x1, w2a_ref[...], preferred_element_type=jnp.float32)
         + jnp.dot(agg1, w2b_ref[...], preferred_element_type=jnp.float32)
         + b2_ref[...])
    h = jnp.maximum(h, 0.0)
    h = h / (jnp.sqrt(jnp.sum(h * h, axis=1, keepdims=True)) + 1e-6)
    x2 = h + x1
    x2lo_ref[...] = x2[:, :D]
    x2hi_ref[...] = x2[:, D:]


def _l3_body(x2lo_ref, x2hi_ref, rlo_ref, rhi_ref, w3alo_ref, w3ahi_ref,
             w3blo_ref, w3bhi_ref, b3_ref, w4p_ref, y_ref):
    agg_lo = rlo_ref[0] + rlo_ref[1]
    agg_hi = rhi_ref[0] + rhi_ref[1]
    h = (jnp.dot(x2lo_ref[...], w3alo_ref[...], preferred_element_type=jnp.float32)
         + jnp.dot(x2hi_ref[...], w3ahi_ref[...], preferred_element_type=jnp.float32)
         + jnp.dot(agg_lo, w3blo_ref[...], preferred_element_type=jnp.float32)
         + jnp.dot(agg_hi, w3bhi_ref[...], preferred_element_type=jnp.float32)
         + b3_ref[...])
    h = jnp.maximum(h, 0.0)
    h = h / (jnp.sqrt(jnp.sum(h * h, axis=1, keepdims=True)) + 1e-6)
    y_ref[...] = jnp.dot(h, w4p_ref[...], preferred_element_type=jnp.float32)


def _row_spec(d):
    return pl.BlockSpec((M_BLK, d), lambda m: (m, 0))


def _pair_spec(d):
    return pl.BlockSpec((NC, M_BLK, d), lambda m: (0, m, 0))


def _full_spec(r, c):
    return pl.BlockSpec((r, c), lambda m: (0, 0))


_l1 = pl.pallas_call(
    _l1_body,
    grid=(GRID_M,),
    in_specs=[_row_spec(D), _pair_spec(D), _full_spec(D, H), _full_spec(D, H)],
    out_specs=[_row_spec(H), _row_spec(D)],
    out_shape=[jax.ShapeDtypeStruct((N, H), jnp.float32),
               jax.ShapeDtypeStruct((N, D), jnp.float32)],
)

_l2 = pl.pallas_call(
    _l2_body,
    grid=(GRID_M,),
    in_specs=[_row_spec(H), _row_spec(D), _pair_spec(D),
              _full_spec(D, H), _full_spec(D, H),
              _full_spec(H, H), _full_spec(H, H), _full_spec(1, H)],
    out_specs=[_row_spec(D), _row_spec(D)],
    out_shape=[jax.ShapeDtypeStruct((N, D), jnp.float32),
               jax.ShapeDtypeStruct((N, D), jnp.float32)],
)

_l3 = pl.pallas_call(
    _l3_body,
    grid=(GRID_M,),
    in_specs=[_row_spec(D), _row_spec(D), _pair_spec(D), _pair_spec(D),
              _full_spec(D, H), _full_spec(D, H), _full_spec(D, H),
              _full_spec(D, H), _full_spec(1, H), _full_spec(H, D)],
    out_specs=_row_spec(D),
    out_shape=jax.ShapeDtypeStruct((N, D), jnp.float32),
)


def kernel(edges, vertices, target_idx, W1, W2, b2, W3, b3, W4):
    src = edges[:, 1].astype(jnp.int32)
    dst = edges[:, 0].astype(jnp.int32)
    pad = NW * E_W - E
    srcx = (jnp.concatenate([src, jnp.zeros((pad,), jnp.int32)])
            .reshape(K_CH, NW, CH).transpose(1, 0, 2).reshape(NW, E_W))
    chunk_off = (jnp.arange(E_W, dtype=jnp.int32) % 8) * N
    srcx = srcx + chunk_off[None, :]
    dstx = (jnp.concatenate([dst, jnp.full((pad,), N, jnp.int32)])
            .reshape(K_CH, NW, CH).transpose(1, 0, 2))

    w1t = W1.T
    w1a, w1b = w1t[:D], w1t[D:]
    w2t = W2.T
    w2a, w2b = w2t[:H], w2t[H:]
    w3t = W3.T
    w3a_lo, w3a_hi = w3t[:D], w3t[D:H]
    w3b_lo, w3b_hi = w3t[H:H + D], w3t[H + D:]
    w4p = jnp.zeros((D, H), jnp.float32).at[:2].set(W4).T

    _seg = _segsum()

    def segsum(t, s, d):
        return _seg(jnp.concatenate([t, t, t, t, t, t, t, t]), s, d)

    p = segsum(vertices, srcx, dstx)
    x1, agg0 = _l1(vertices, p, w1a, w1b)
    q = segsum(agg0, srcx, dstx)
    x2lo, x2hi = _l2(x1, agg0, q, w1a, w1b, w2a, w2b, b2.reshape(1, H))
    rlo = segsum(x2lo, srcx, dstx)
    rhi = segsum(x2hi, srcx, dstx)
    y = _l3(x2lo, x2hi, rlo, rhi, w3a_lo, w3a_hi, w3b_lo, w3b_hi,
            b3.reshape(1, H), w4p)
    yt = _target_gather()(y, target_idx.astype(jnp.int32))
    return yt[:, :2][None, :, :]
